# baseline (speedup 1.0000x reference)
import functools

import jax
import jax.numpy as jnp
from jax import lax
from jax.experimental import pallas as pl
from jax.experimental.pallas import tpu as pltpu
from jax.experimental.pallas import tpu_sc as plsc

_V = 1000000
_B = 4096
_S = 200
_DIM = 64
_PDIM = 128
_NFLAT = _B * _S
_NW = 32
_CH = 128
_NCH_TOTAL = _NFLAT // _CH
_NCH_PER_W = _NCH_TOTAL // _NW
_TOK_PER_W = _NCH_PER_W * _CH
_K = 4
_NSC = _NCH_PER_W // _K

_mesh = plsc.VectorSubcoreMesh(core_axis_name="c", subcore_axis_name="s")


@functools.partial(
    pl.kernel,
    mesh=_mesh,
    out_type=jax.ShapeDtypeStruct((_NCH_TOTAL, _CH, _PDIM), jnp.float32),
    scratch_types=[
        pltpu.VMEM((_TOK_PER_W,), jnp.int32),
        pltpu.VMEM((_K, _CH, _DIM), jnp.float32),
        pltpu.VMEM((_K, _CH, _DIM), jnp.float32),
        pltpu.SemaphoreType.DMA,
        pltpu.SemaphoreType.DMA,
        pltpu.SemaphoreType.DMA,
        pltpu.SemaphoreType.DMA,
    ],
    compiler_params=pltpu.CompilerParams(use_tc_tiling_on_sc=False),
)
def _emb_lookup(idx_hbm, table_hbm, out_hbm, idx_v, rows0, rows1,
                gsem0, gsem1, osem0, osem1):
    nc = plsc.get_sparse_core_info().num_cores
    wid = lax.axis_index("s") * nc + lax.axis_index("c")
    base = wid * _NCH_PER_W
    rows = (rows0, rows1)
    gsem = (gsem0, gsem1)
    osem = (osem0, osem1)

    pltpu.sync_copy(idx_hbm.at[pl.ds(wid * _TOK_PER_W, _TOK_PER_W)], idx_v)

    def fire(i, b):
        for j in range(_K):
            pltpu.make_async_copy(
                table_hbm.at[idx_v.at[pl.ds((i * _K + j) * _CH, _CH)]],
                rows[b].at[j],
                gsem[b],
            ).start()

    def wait_gathers(i, b):
        for j in range(_K):
            pltpu.make_async_copy(
                table_hbm.at[idx_v.at[pl.ds((i * _K + j) * _CH, _CH)]],
                rows[b].at[j],
                gsem[b],
            ).wait()

    def start_out(i, b):
        pltpu.make_async_copy(
            rows[b],
            out_hbm.at[pl.ds(base + i * _K, _K), :, pl.ds(0, _DIM)],
            osem[b],
        ).start()

    def wait_out(i, b):
        pltpu.make_async_copy(
            rows[b],
            out_hbm.at[pl.ds(base + i * _K, _K), :, pl.ds(0, _DIM)],
            osem[b],
        ).wait()

    fire(0, 0)
    fire(1, 1)
    wait_gathers(0, 0)
    start_out(0, 0)

    def body(u, carry):
        for b in range(2):
            i = 2 + 2 * u + b
            wait_out(i - 2, b)
            fire(i, b)
            wait_gathers(i - 1, 1 - b)
            start_out(i - 1, 1 - b)
        return carry

    lax.fori_loop(0, (_NSC - 2) // 2, body, 0)

    last = (_NSC - 1) % 2
    wait_gathers(_NSC - 1, last)
    start_out(_NSC - 1, last)
    wait_out(_NSC - 2, 1 - last)
    wait_out(_NSC - 1, last)


def kernel(token_idx_list, embedding):
    idx = token_idx_list.astype(jnp.int32).reshape(_NFLAT)
    table = lax.optimization_barrier(
        embedding.reshape(_V // 2, 2 * _DIM)
    ).reshape(_V, _DIM)
    out = _emb_lookup(idx, table)
    return out.reshape(_NFLAT, _PDIM)[:, :_DIM].reshape(_B, _S, _DIM)
